# baseline (speedup 1.0000x reference)
import functools

import jax
import jax.numpy as jnp
from jax import lax
from jax.experimental import pallas as pl
from jax.experimental.pallas import tpu as pltpu
from jax.experimental.pallas import tpu_sc as plsc

N = 10000
E = 320000
D = 128

NC = 2
NS = 16
NW = NC * NS
EPT = E // NW
K = 80
NCHUNK = EPT // K
NP = 10240
RPT = NP // NS

_MESH = plsc.VectorSubcoreMesh(core_axis_name="c", subcore_axis_name="s")


NPT = NP // NS


def _deg_body(dst2_hbm, zdeg_hbm, ones_hbm, out_hbm, deg_sh, didx_v, ones_v, sem):
    c = lax.axis_index("c")
    s = lax.axis_index("s")
    wid = s * NC + c

    pltpu.sync_copy(zdeg_hbm, deg_sh.at[pl.ds(s * NPT, NPT)])
    pltpu.sync_copy(dst2_hbm.at[wid], didx_v)
    pltpu.sync_copy(ones_hbm, ones_v)
    plsc.subcore_barrier()

    GRP = 5

    for b in range(GRP):
        pltpu.async_copy(ones_v, deg_sh.at[didx_v.at[b]], sem, add=True)

    def fire(g, carry):
        for b in range(GRP):
            pltpu.async_copy(ones_v, deg_sh.at[didx_v.at[g * GRP + b]], sem,
                             add=True)
        for b in range(GRP):
            pltpu.make_async_copy(ones_v,
                                  deg_sh.at[didx_v.at[(g - 1) * GRP + b]],
                                  sem).wait()
        return carry

    lax.fori_loop(1, NCHUNK // GRP, fire, 0)

    last = NCHUNK // GRP - 1
    for b in range(GRP):
        pltpu.make_async_copy(ones_v, deg_sh.at[didx_v.at[last * GRP + b]],
                              sem).wait()
    plsc.subcore_barrier()

    pltpu.sync_copy(deg_sh.at[pl.ds(s * NPT, NPT)],
                    out_hbm.at[c, pl.ds(s * NPT, NPT)])


_deg_call = pl.kernel(
    _deg_body,
    out_type=jax.ShapeDtypeStruct((NC, NP), jnp.float32),
    mesh=_MESH,
    scratch_types=[
        pltpu.VMEM_SHARED((NP,), jnp.float32),
        pltpu.VMEM((NCHUNK, K), jnp.int32),
        pltpu.VMEM((K,), jnp.float32),
        pltpu.SemaphoreType.DMA,
    ],
)


NBANK = 3


def _seg_body(y_hbm, src1_hbm, dst1_hbm, zeros_hbm, out_hbm,
              acc_sh, sidx_v, didx0, didx1, didx2, rows_v,
              isem0, isem1, isem2, gsem0, gsem1, gsem2, ssem0, ssem1, ssem2):
    c = lax.axis_index("c")
    s = lax.axis_index("s")
    wid = s * NC + c
    didxs = (didx0, didx1, didx2)
    isems = (isem0, isem1, isem2)
    gsems = (gsem0, gsem1, gsem2)
    ssems = (ssem0, ssem1, ssem2)

    @pl.when(jnp.logical_and(c == 0, s < NS - 1))
    def _():
        pltpu.sync_copy(y_hbm.at[pl.ds(s * RPT, RPT)],
                        acc_sh.at[pl.ds(s * RPT, RPT)])

    @pl.when(jnp.logical_and(c == 0, s == NS - 1))
    def _():
        pltpu.sync_copy(y_hbm.at[pl.ds((NS - 1) * RPT, N - (NS - 1) * RPT)],
                        acc_sh.at[pl.ds((NS - 1) * RPT, N - (NS - 1) * RPT)])
        pltpu.sync_copy(zeros_hbm.at[pl.ds(0, NP - N)],
                        acc_sh.at[pl.ds(N, NP - N)])

    @pl.when(c == 1)
    def _():
        pltpu.sync_copy(zeros_hbm, acc_sh.at[pl.ds(s * RPT, RPT)])

    pltpu.sync_copy(src1_hbm.at[wid], sidx_v)
    plsc.subcore_barrier()

    ebase = wid * EPT

    def fire_didx(i, b):
        pltpu.async_copy(dst1_hbm.at[pl.ds(ebase + i * K, K)], didxs[b],
                         isems[b])

    def wait_didx(i, b):
        pltpu.make_async_copy(dst1_hbm.at[pl.ds(ebase + i * K, K)], didxs[b],
                              isems[b]).wait()

    def fire_gather(i, b):
        pltpu.async_copy(y_hbm.at[sidx_v.at[pl.ds(i * K, K)]], rows_v.at[b],
                         gsems[b])

    def wait_gather(i, b):
        pltpu.make_async_copy(y_hbm.at[sidx_v.at[pl.ds(i * K, K)]],
                              rows_v.at[b], gsems[b]).wait()

    def fire_scatter(b):
        pltpu.async_copy(rows_v.at[b], acc_sh.at[didxs[b]], ssems[b],
                         add=True)

    def drain_scatter(b):
        pltpu.make_async_copy(rows_v.at[b], acc_sh.at[didxs[b]],
                              ssems[b]).wait()

    def body(i, t):
        b = t
        b2 = (t + 1) % NBANK
        drain_scatter(b)
        fire_didx(i, b)
        fire_gather(i, b)
        wait_gather(i - 2, b2)
        wait_didx(i - 2, b2)
        fire_scatter(b2)

    for i in range(NBANK):
        fire_didx(i, i)
        fire_gather(i, i)
    wait_gather(0, 0)
    wait_didx(0, 0)
    fire_scatter(0)

    def three_chunks(q, carry):
        body(3 * q, 0)
        body(3 * q + 1, 1)
        body(3 * q + 2, 2)
        return carry

    lax.fori_loop(1, (NCHUNK - 2) // NBANK, three_chunks, 0)

    body(NCHUNK - 2, 0)
    body(NCHUNK - 1, 1)
    wait_gather(NCHUNK - 2, 0)
    wait_didx(NCHUNK - 2, 0)
    fire_scatter(0)
    wait_gather(NCHUNK - 1, 1)
    wait_didx(NCHUNK - 1, 1)
    fire_scatter(1)
    drain_scatter(2)
    drain_scatter(0)
    drain_scatter(1)
    plsc.subcore_barrier()

    pltpu.sync_copy(acc_sh.at[pl.ds(s * RPT, RPT)],
                    out_hbm.at[c, pl.ds(s * RPT, RPT)])


_seg_call = pl.kernel(
    _seg_body,
    out_type=jax.ShapeDtypeStruct((NC, NP, D), jnp.float32),
    mesh=_MESH,
    scratch_types=[
        pltpu.VMEM_SHARED((NP, D), jnp.float32),
        pltpu.VMEM((EPT,), jnp.int32),
        pltpu.VMEM((K,), jnp.int32),
        pltpu.VMEM((K,), jnp.int32),
        pltpu.VMEM((K,), jnp.int32),
        pltpu.VMEM((NBANK, K, D), jnp.float32),
        pltpu.SemaphoreType.DMA,
        pltpu.SemaphoreType.DMA,
        pltpu.SemaphoreType.DMA,
        pltpu.SemaphoreType.DMA,
        pltpu.SemaphoreType.DMA,
        pltpu.SemaphoreType.DMA,
        pltpu.SemaphoreType.DMA,
        pltpu.SemaphoreType.DMA,
        pltpu.SemaphoreType.DMA,
    ],
)


R = 1000


def _y_body(part_ref, x_ref, w_ref, y_ref, dis_ref, dis_sc):
    i = pl.program_id(0)

    @pl.when(i == 0)
    def _():
        deg = jnp.sum(part_ref[...], axis=0) + 1.0
        full = lax.rsqrt(deg).reshape(NP, 1)
        dis_sc[...] = full
        dis_ref[...] = full

    xw = jnp.dot(x_ref[...], w_ref[...], preferred_element_type=jnp.float32)
    y_ref[...] = xw * dis_sc[pl.ds(i * R, R)]


_y_call = pl.pallas_call(
    _y_body,
    grid=(N // R,),
    in_specs=[
        pl.BlockSpec((NC, NP), lambda i: (0, 0)),
        pl.BlockSpec((R, D), lambda i: (i, 0)),
        pl.BlockSpec((D, D), lambda i: (0, 0)),
    ],
    out_specs=[
        pl.BlockSpec((R, D), lambda i: (i, 0)),
        pl.BlockSpec((NP, 1), lambda i: (0, 0)),
    ],
    out_shape=[
        jax.ShapeDtypeStruct((N, D), jnp.float32),
        jax.ShapeDtypeStruct((NP, 1), jnp.float32),
    ],
    scratch_shapes=[pltpu.VMEM((NP, 1), jnp.float32)],
)


def _cm_body(acc_ref, dis_ref, b_ref, w_ref, y2_ref):
    h = (acc_ref[0] + acc_ref[1]) * dis_ref[...] + b_ref[...]
    h = jnp.maximum(h, 0.0)
    hw = jnp.dot(h, w_ref[...], preferred_element_type=jnp.float32)
    y2_ref[...] = hw * dis_ref[...]


_cm_call = pl.pallas_call(
    _cm_body,
    grid=(N // R,),
    in_specs=[
        pl.BlockSpec((NC, R, D), lambda i: (0, i, 0)),
        pl.BlockSpec((R, 1), lambda i: (i, 0)),
        pl.BlockSpec((1, D), lambda i: (0, 0)),
        pl.BlockSpec((D, D), lambda i: (0, 0)),
    ],
    out_specs=pl.BlockSpec((R, D), lambda i: (i, 0)),
    out_shape=jax.ShapeDtypeStruct((N, D), jnp.float32),
)


def _fin_body(acc_ref, dis_ref, b_ref, out_ref):
    h = (acc_ref[0] + acc_ref[1]) * dis_ref[...] + b_ref[...]
    out_ref[...] = jnp.maximum(h, 0.0)


_fin_call = pl.pallas_call(
    _fin_body,
    grid=(N // R,),
    in_specs=[
        pl.BlockSpec((NC, R, D), lambda i: (0, i, 0)),
        pl.BlockSpec((R, 1), lambda i: (i, 0)),
        pl.BlockSpec((1, D), lambda i: (0, 0)),
    ],
    out_specs=pl.BlockSpec((R, D), lambda i: (i, 0)),
    out_shape=jax.ShapeDtypeStruct((N, D), jnp.float32),
)


def kernel(x, edge_index, W1, b1, W2, b2):
    ei = edge_index.astype(jnp.int32)
    src1 = ei[0].reshape(NW, EPT)
    dst2 = ei[1].reshape(NW, NCHUNK, K)
    zeros_tile = jnp.zeros((RPT, D), jnp.float32)
    zeros_deg = jnp.zeros((NPT,), jnp.float32)
    ones_k = jnp.ones((K,), jnp.float32)

    part = _deg_call(dst2, zeros_deg, ones_k)
    y1, dis_col = _y_call(part, x, W1)
    dst1 = ei[1].reshape(NW, EPT)
    acc1 = _seg_call(y1, src1, dst1.reshape(E), zeros_tile)
    y2 = _cm_call(acc1, dis_col, b1.reshape(1, D), W2)
    acc2 = _seg_call(y2, src1, dst1.reshape(E), zeros_tile)
    return _fin_call(acc2, dis_col, b2.reshape(1, D))


# speedup vs baseline: 1.0436x; 1.0436x over previous
import functools

import jax
import jax.numpy as jnp
from jax import lax
from jax.experimental import pallas as pl
from jax.experimental.pallas import tpu as pltpu
from jax.experimental.pallas import tpu_sc as plsc

N = 10000
E = 320000
D = 128

NC = 2
NS = 16
NW = NC * NS
EPT = E // NW
K = 80
NCHUNK = EPT // K
NP = 10240
RPT = NP // NS

_MESH = plsc.VectorSubcoreMesh(core_axis_name="c", subcore_axis_name="s")


NPT = NP // NS


def _deg_body(dst2_hbm, zdeg_hbm, ones_hbm, out_hbm, deg_sh, didx_v, ones_v, sem):
    c = lax.axis_index("c")
    s = lax.axis_index("s")
    wid = s * NC + c

    pltpu.sync_copy(zdeg_hbm, deg_sh.at[pl.ds(s * NPT, NPT)])
    pltpu.sync_copy(dst2_hbm.at[wid], didx_v)
    pltpu.sync_copy(ones_hbm, ones_v)
    plsc.subcore_barrier()

    GRP = 5

    for b in range(GRP):
        pltpu.async_copy(ones_v, deg_sh.at[didx_v.at[b]], sem, add=True)

    def fire(g, carry):
        for b in range(GRP):
            pltpu.async_copy(ones_v, deg_sh.at[didx_v.at[g * GRP + b]], sem,
                             add=True)
        for b in range(GRP):
            pltpu.make_async_copy(ones_v,
                                  deg_sh.at[didx_v.at[(g - 1) * GRP + b]],
                                  sem).wait()
        return carry

    lax.fori_loop(1, NCHUNK // GRP, fire, 0)

    last = NCHUNK // GRP - 1
    for b in range(GRP):
        pltpu.make_async_copy(ones_v, deg_sh.at[didx_v.at[last * GRP + b]],
                              sem).wait()
    plsc.subcore_barrier()

    pltpu.sync_copy(deg_sh.at[pl.ds(s * NPT, NPT)],
                    out_hbm.at[c, pl.ds(s * NPT, NPT)])


_deg_call = pl.kernel(
    _deg_body,
    out_type=jax.ShapeDtypeStruct((NC, NP), jnp.float32),
    mesh=_MESH,
    scratch_types=[
        pltpu.VMEM_SHARED((NP,), jnp.float32),
        pltpu.VMEM((NCHUNK, K), jnp.int32),
        pltpu.VMEM((K,), jnp.float32),
        pltpu.SemaphoreType.DMA,
    ],
)


NBANK = 3


def _seg_body(y_hbm, src1_hbm, dst1_hbm, zeros_hbm, out_hbm,
              acc_sh, sidx_v, didx0, didx1, didx2, rows_v,
              isem0, isem1, isem2, gsem0, gsem1, gsem2, ssem0, ssem1, ssem2):
    c = lax.axis_index("c")
    s = lax.axis_index("s")
    wid = s * NC + c
    didxs = (didx0, didx1, didx2)
    isems = (isem0, isem1, isem2)
    gsems = (gsem0, gsem1, gsem2)
    ssems = (ssem0, ssem1, ssem2)

    @pl.when(jnp.logical_and(c == 0, s < NS - 1))
    def _():
        pltpu.sync_copy(y_hbm.at[pl.ds(s * RPT, RPT)],
                        acc_sh.at[pl.ds(s * RPT, RPT)])

    @pl.when(jnp.logical_and(c == 0, s == NS - 1))
    def _():
        pltpu.sync_copy(y_hbm.at[pl.ds((NS - 1) * RPT, N - (NS - 1) * RPT)],
                        acc_sh.at[pl.ds((NS - 1) * RPT, N - (NS - 1) * RPT)])
        pltpu.sync_copy(zeros_hbm.at[pl.ds(0, NP - N)],
                        acc_sh.at[pl.ds(N, NP - N)])

    @pl.when(c == 1)
    def _():
        pltpu.sync_copy(zeros_hbm, acc_sh.at[pl.ds(s * RPT, RPT)])

    pltpu.sync_copy(src1_hbm.at[wid], sidx_v)
    plsc.subcore_barrier()

    ebase = wid * EPT

    def fire_didx(i, b):
        pltpu.async_copy(dst1_hbm.at[pl.ds(ebase + i * K, K)], didxs[b],
                         isems[b])

    def wait_didx(i, b):
        pltpu.make_async_copy(dst1_hbm.at[pl.ds(ebase + i * K, K)], didxs[b],
                              isems[b]).wait()

    def fire_gather(i, b):
        pltpu.async_copy(y_hbm.at[sidx_v.at[pl.ds(i * K, K)]], rows_v.at[b],
                         gsems[b])

    def wait_gather(i, b):
        pltpu.make_async_copy(y_hbm.at[sidx_v.at[pl.ds(i * K, K)]],
                              rows_v.at[b], gsems[b]).wait()

    def fire_scatter(b):
        pltpu.async_copy(rows_v.at[b], acc_sh.at[didxs[b]], ssems[b],
                         add=True)

    def drain_scatter(b):
        pltpu.make_async_copy(rows_v.at[b], acc_sh.at[didxs[b]],
                              ssems[b]).wait()

    def body(i, t):
        b = t
        b2 = (t + 1) % NBANK
        drain_scatter(b)
        fire_didx(i, b)
        fire_gather(i, b)
        wait_gather(i - 2, b2)
        wait_didx(i - 2, b2)
        fire_scatter(b2)

    for i in range(NBANK):
        fire_didx(i, i)
        fire_gather(i, i)
    wait_gather(0, 0)
    wait_didx(0, 0)
    fire_scatter(0)

    def three_chunks(q, carry):
        body(3 * q, 0)
        body(3 * q + 1, 1)
        body(3 * q + 2, 2)
        return carry

    lax.fori_loop(1, (NCHUNK - 2) // NBANK, three_chunks, 0)

    body(NCHUNK - 2, 0)
    body(NCHUNK - 1, 1)
    wait_gather(NCHUNK - 2, 0)
    wait_didx(NCHUNK - 2, 0)
    fire_scatter(0)
    wait_gather(NCHUNK - 1, 1)
    wait_didx(NCHUNK - 1, 1)
    fire_scatter(1)
    drain_scatter(2)
    drain_scatter(0)
    drain_scatter(1)
    plsc.subcore_barrier()

    pltpu.sync_copy(acc_sh.at[pl.ds(s * RPT, RPT)],
                    out_hbm.at[c, pl.ds(s * RPT, RPT)])


_seg_call = pl.kernel(
    _seg_body,
    out_type=jax.ShapeDtypeStruct((NC, NP, D), jnp.float32),
    mesh=_MESH,
    scratch_types=[
        pltpu.VMEM_SHARED((NP, D), jnp.float32),
        pltpu.VMEM((EPT,), jnp.int32),
        pltpu.VMEM((K,), jnp.int32),
        pltpu.VMEM((K,), jnp.int32),
        pltpu.VMEM((K,), jnp.int32),
        pltpu.VMEM((NBANK, K, D), jnp.float32),
        pltpu.SemaphoreType.DMA,
        pltpu.SemaphoreType.DMA,
        pltpu.SemaphoreType.DMA,
        pltpu.SemaphoreType.DMA,
        pltpu.SemaphoreType.DMA,
        pltpu.SemaphoreType.DMA,
        pltpu.SemaphoreType.DMA,
        pltpu.SemaphoreType.DMA,
        pltpu.SemaphoreType.DMA,
    ],
)


R = 5000


def _y_body(part_ref, x_ref, w_ref, y_ref, dis_ref, dis_sc):
    i = pl.program_id(0)

    @pl.when(i == 0)
    def _():
        deg = jnp.sum(part_ref[...], axis=0) + 1.0
        full = lax.rsqrt(deg).reshape(NP, 1)
        dis_sc[...] = full
        dis_ref[...] = full

    xw = jnp.dot(x_ref[...], w_ref[...], preferred_element_type=jnp.float32)
    y_ref[...] = xw * dis_sc[pl.ds(i * R, R)]


_y_call = pl.pallas_call(
    _y_body,
    grid=(N // R,),
    in_specs=[
        pl.BlockSpec((NC, NP), lambda i: (0, 0)),
        pl.BlockSpec((R, D), lambda i: (i, 0)),
        pl.BlockSpec((D, D), lambda i: (0, 0)),
    ],
    out_specs=[
        pl.BlockSpec((R, D), lambda i: (i, 0)),
        pl.BlockSpec((NP, 1), lambda i: (0, 0)),
    ],
    out_shape=[
        jax.ShapeDtypeStruct((N, D), jnp.float32),
        jax.ShapeDtypeStruct((NP, 1), jnp.float32),
    ],
    scratch_shapes=[pltpu.VMEM((NP, 1), jnp.float32)],
)


def _cm_body(acc_ref, dis_ref, b_ref, w_ref, y2_ref):
    h = (acc_ref[0] + acc_ref[1]) * dis_ref[...] + b_ref[...]
    h = jnp.maximum(h, 0.0)
    hw = jnp.dot(h, w_ref[...], preferred_element_type=jnp.float32)
    y2_ref[...] = hw * dis_ref[...]


_cm_call = pl.pallas_call(
    _cm_body,
    grid=(N // R,),
    in_specs=[
        pl.BlockSpec((NC, R, D), lambda i: (0, i, 0)),
        pl.BlockSpec((R, 1), lambda i: (i, 0)),
        pl.BlockSpec((1, D), lambda i: (0, 0)),
        pl.BlockSpec((D, D), lambda i: (0, 0)),
    ],
    out_specs=pl.BlockSpec((R, D), lambda i: (i, 0)),
    out_shape=jax.ShapeDtypeStruct((N, D), jnp.float32),
)


def _fin_body(acc_ref, dis_ref, b_ref, out_ref):
    h = (acc_ref[0] + acc_ref[1]) * dis_ref[...] + b_ref[...]
    out_ref[...] = jnp.maximum(h, 0.0)


_fin_call = pl.pallas_call(
    _fin_body,
    grid=(N // R,),
    in_specs=[
        pl.BlockSpec((NC, R, D), lambda i: (0, i, 0)),
        pl.BlockSpec((R, 1), lambda i: (i, 0)),
        pl.BlockSpec((1, D), lambda i: (0, 0)),
    ],
    out_specs=pl.BlockSpec((R, D), lambda i: (i, 0)),
    out_shape=jax.ShapeDtypeStruct((N, D), jnp.float32),
)


def kernel(x, edge_index, W1, b1, W2, b2):
    ei = edge_index.astype(jnp.int32)
    src1 = ei[0].reshape(NW, EPT)
    dst2 = ei[1].reshape(NW, NCHUNK, K)
    zeros_tile = jnp.zeros((RPT, D), jnp.float32)
    zeros_deg = jnp.zeros((NPT,), jnp.float32)
    ones_k = jnp.ones((K,), jnp.float32)

    part = _deg_call(dst2, zeros_deg, ones_k)
    y1, dis_col = _y_call(part, x, W1)
    dst1 = ei[1].reshape(NW, EPT)
    acc1 = _seg_call(y1, src1, dst1.reshape(E), zeros_tile)
    y2 = _cm_call(acc1, dis_col, b1.reshape(1, D), W2)
    acc2 = _seg_call(y2, src1, dst1.reshape(E), zeros_tile)
    return _fin_call(acc2, dis_col, b2.reshape(1, D))


# speedup vs baseline: 1.0437x; 1.0001x over previous
import jax
import jax.numpy as jnp
from jax import lax
from jax.experimental import pallas as pl
from jax.experimental.pallas import tpu as pltpu
from jax.experimental.pallas import tpu_sc as plsc

N = 10000
E = 320000
D = 128

NC = 2
NS = 16
NW = NC * NS
EPT = E // NW
K = 80
NCHUNK = EPT // K
NP = 10240
RPT = NP // NS

_MESH = plsc.VectorSubcoreMesh(core_axis_name="c", subcore_axis_name="s")


NPT = NP // NS


def _deg_body(dst2_hbm, zdeg_hbm, ones_hbm, out_hbm, deg_sh, didx_v, ones_v, sem):
    c = lax.axis_index("c")
    s = lax.axis_index("s")
    wid = s * NC + c

    pltpu.sync_copy(zdeg_hbm, deg_sh.at[pl.ds(s * NPT, NPT)])
    pltpu.sync_copy(dst2_hbm.at[wid], didx_v)
    pltpu.sync_copy(ones_hbm, ones_v)
    plsc.subcore_barrier()

    GRP = 5

    for b in range(GRP):
        pltpu.async_copy(ones_v, deg_sh.at[didx_v.at[b]], sem, add=True)

    def fire(g, carry):
        for b in range(GRP):
            pltpu.async_copy(ones_v, deg_sh.at[didx_v.at[g * GRP + b]], sem,
                             add=True)
        for b in range(GRP):
            pltpu.make_async_copy(ones_v,
                                  deg_sh.at[didx_v.at[(g - 1) * GRP + b]],
                                  sem).wait()
        return carry

    lax.fori_loop(1, NCHUNK // GRP, fire, 0)

    last = NCHUNK // GRP - 1
    for b in range(GRP):
        pltpu.make_async_copy(ones_v, deg_sh.at[didx_v.at[last * GRP + b]],
                              sem).wait()
    plsc.subcore_barrier()

    pltpu.sync_copy(deg_sh.at[pl.ds(s * NPT, NPT)],
                    out_hbm.at[c, pl.ds(s * NPT, NPT)])


_deg_call = pl.kernel(
    _deg_body,
    out_type=jax.ShapeDtypeStruct((NC, NP), jnp.float32),
    mesh=_MESH,
    scratch_types=[
        pltpu.VMEM_SHARED((NP,), jnp.float32),
        pltpu.VMEM((NCHUNK, K), jnp.int32),
        pltpu.VMEM((K,), jnp.float32),
        pltpu.SemaphoreType.DMA,
    ],
)


NBANK = 3


def _seg_body(y_hbm, src1_hbm, dst1_hbm, zeros_hbm, out_hbm,
              acc_sh, sidx_v, didx0, didx1, didx2, rows_v,
              isem0, isem1, isem2, gsem0, gsem1, gsem2, ssem0, ssem1, ssem2):
    c = lax.axis_index("c")
    s = lax.axis_index("s")
    wid = s * NC + c
    didxs = (didx0, didx1, didx2)
    isems = (isem0, isem1, isem2)
    gsems = (gsem0, gsem1, gsem2)
    ssems = (ssem0, ssem1, ssem2)

    @pl.when(jnp.logical_and(c == 0, s < NS - 1))
    def _():
        pltpu.sync_copy(y_hbm.at[pl.ds(s * RPT, RPT)],
                        acc_sh.at[pl.ds(s * RPT, RPT)])

    @pl.when(jnp.logical_and(c == 0, s == NS - 1))
    def _():
        pltpu.sync_copy(y_hbm.at[pl.ds((NS - 1) * RPT, N - (NS - 1) * RPT)],
                        acc_sh.at[pl.ds((NS - 1) * RPT, N - (NS - 1) * RPT)])
        pltpu.sync_copy(zeros_hbm.at[pl.ds(0, NP - N)],
                        acc_sh.at[pl.ds(N, NP - N)])

    @pl.when(c == 1)
    def _():
        pltpu.sync_copy(zeros_hbm, acc_sh.at[pl.ds(s * RPT, RPT)])

    pltpu.sync_copy(src1_hbm.at[wid], sidx_v)
    plsc.subcore_barrier()

    ebase = wid * EPT

    def fire_didx(i, b):
        pltpu.async_copy(dst1_hbm.at[pl.ds(ebase + i * K, K)], didxs[b],
                         isems[b])

    def wait_didx(i, b):
        pltpu.make_async_copy(dst1_hbm.at[pl.ds(ebase + i * K, K)], didxs[b],
                              isems[b]).wait()

    def fire_gather(i, b):
        pltpu.async_copy(y_hbm.at[sidx_v.at[pl.ds(i * K, K)]], rows_v.at[b],
                         gsems[b])

    def wait_gather(i, b):
        pltpu.make_async_copy(y_hbm.at[sidx_v.at[pl.ds(i * K, K)]],
                              rows_v.at[b], gsems[b]).wait()

    def fire_scatter(b):
        pltpu.async_copy(rows_v.at[b], acc_sh.at[didxs[b]], ssems[b],
                         add=True)

    def drain_scatter(b):
        pltpu.make_async_copy(rows_v.at[b], acc_sh.at[didxs[b]],
                              ssems[b]).wait()

    def body(i, t):
        b = t
        b2 = (t + 1) % NBANK
        drain_scatter(b)
        fire_didx(i, b)
        fire_gather(i, b)
        wait_gather(i - 2, b2)
        wait_didx(i - 2, b2)
        fire_scatter(b2)

    for i in range(NBANK):
        fire_didx(i, i)
        fire_gather(i, i)
    wait_gather(0, 0)
    wait_didx(0, 0)
    fire_scatter(0)

    def three_chunks(q, carry):
        body(3 * q, 0)
        body(3 * q + 1, 1)
        body(3 * q + 2, 2)
        return carry

    lax.fori_loop(1, (NCHUNK - 2) // NBANK, three_chunks, 0)

    body(NCHUNK - 2, 0)
    body(NCHUNK - 1, 1)
    wait_gather(NCHUNK - 2, 0)
    wait_didx(NCHUNK - 2, 0)
    fire_scatter(0)
    wait_gather(NCHUNK - 1, 1)
    wait_didx(NCHUNK - 1, 1)
    fire_scatter(1)
    drain_scatter(2)
    drain_scatter(0)
    drain_scatter(1)
    plsc.subcore_barrier()

    pltpu.sync_copy(acc_sh.at[pl.ds(s * RPT, RPT)],
                    out_hbm.at[c, pl.ds(s * RPT, RPT)])


_seg_call = pl.kernel(
    _seg_body,
    out_type=jax.ShapeDtypeStruct((NC, NP, D), jnp.float32),
    mesh=_MESH,
    scratch_types=[
        pltpu.VMEM_SHARED((NP, D), jnp.float32),
        pltpu.VMEM((EPT,), jnp.int32),
        pltpu.VMEM((K,), jnp.int32),
        pltpu.VMEM((K,), jnp.int32),
        pltpu.VMEM((K,), jnp.int32),
        pltpu.VMEM((NBANK, K, D), jnp.float32),
        pltpu.SemaphoreType.DMA,
        pltpu.SemaphoreType.DMA,
        pltpu.SemaphoreType.DMA,
        pltpu.SemaphoreType.DMA,
        pltpu.SemaphoreType.DMA,
        pltpu.SemaphoreType.DMA,
        pltpu.SemaphoreType.DMA,
        pltpu.SemaphoreType.DMA,
        pltpu.SemaphoreType.DMA,
    ],
)


R = 5000


def _y_body(part_ref, x_ref, w_ref, y_ref, dis_ref, dis_sc):
    i = pl.program_id(0)

    @pl.when(i == 0)
    def _():
        deg = jnp.sum(part_ref[...], axis=0) + 1.0
        full = lax.rsqrt(deg).reshape(NP, 1)
        dis_sc[...] = full
        dis_ref[...] = full

    xw = jnp.dot(x_ref[...], w_ref[...], preferred_element_type=jnp.float32)
    y_ref[...] = xw * dis_sc[pl.ds(i * R, R)]


_y_call = pl.pallas_call(
    _y_body,
    grid=(N // R,),
    in_specs=[
        pl.BlockSpec((NC, NP), lambda i: (0, 0)),
        pl.BlockSpec((R, D), lambda i: (i, 0)),
        pl.BlockSpec((D, D), lambda i: (0, 0)),
    ],
    out_specs=[
        pl.BlockSpec((R, D), lambda i: (i, 0)),
        pl.BlockSpec((NP, 1), lambda i: (0, 0)),
    ],
    out_shape=[
        jax.ShapeDtypeStruct((N, D), jnp.float32),
        jax.ShapeDtypeStruct((NP, 1), jnp.float32),
    ],
    scratch_shapes=[pltpu.VMEM((NP, 1), jnp.float32)],
)


def _cm_body(acc_ref, dis_ref, b_ref, w_ref, y2_ref):
    h = (acc_ref[0] + acc_ref[1]) * dis_ref[...] + b_ref[...]
    h = jnp.maximum(h, 0.0)
    hw = jnp.dot(h, w_ref[...], preferred_element_type=jnp.float32)
    y2_ref[...] = hw * dis_ref[...]


_cm_call = pl.pallas_call(
    _cm_body,
    grid=(N // R,),
    in_specs=[
        pl.BlockSpec((NC, R, D), lambda i: (0, i, 0)),
        pl.BlockSpec((R, 1), lambda i: (i, 0)),
        pl.BlockSpec((1, D), lambda i: (0, 0)),
        pl.BlockSpec((D, D), lambda i: (0, 0)),
    ],
    out_specs=pl.BlockSpec((R, D), lambda i: (i, 0)),
    out_shape=jax.ShapeDtypeStruct((N, D), jnp.float32),
)


def _fin_body(acc_ref, dis_ref, b_ref, out_ref):
    h = (acc_ref[0] + acc_ref[1]) * dis_ref[...] + b_ref[...]
    out_ref[...] = jnp.maximum(h, 0.0)


_fin_call = pl.pallas_call(
    _fin_body,
    grid=(N // R,),
    in_specs=[
        pl.BlockSpec((NC, R, D), lambda i: (0, i, 0)),
        pl.BlockSpec((R, 1), lambda i: (i, 0)),
        pl.BlockSpec((1, D), lambda i: (0, 0)),
    ],
    out_specs=pl.BlockSpec((R, D), lambda i: (i, 0)),
    out_shape=jax.ShapeDtypeStruct((N, D), jnp.float32),
)


def kernel(x, edge_index, W1, b1, W2, b2):
    ei = edge_index.astype(jnp.int32)
    src1 = ei[0].reshape(NW, EPT)
    dst2 = ei[1].reshape(NW, NCHUNK, K)
    zeros_tile = jnp.zeros((RPT, D), jnp.float32)
    zeros_deg = jnp.zeros((NPT,), jnp.float32)
    ones_k = jnp.ones((K,), jnp.float32)

    part = _deg_call(dst2, zeros_deg, ones_k)
    y1, dis_col = _y_call(part, x, W1)
    acc1 = _seg_call(y1, src1, ei[1], zeros_tile)
    y2 = _cm_call(acc1, dis_col, b1.reshape(1, D), W2)
    acc2 = _seg_call(y2, src1, ei[1], zeros_tile)
    return _fin_call(acc2, dis_col, b2.reshape(1, D))
